# baseline (speedup 1.0000x reference)
import functools

import jax
import jax.numpy as jnp
from jax import lax
from jax.experimental import pallas as pl
from jax.experimental.pallas import tpu as pltpu
from jax.experimental.pallas import tpu_sc as plsc

_B = 16384
_D = 128
_NC = 2
_NS = 16
_L = 16
_NW = _NC * _NS
_BPW = _B // _NW
_CH = 128
_NCHUNK = _BPW // _CH

_RSQRT_MAGIC = 0x5F3759DF


def _vsqrt(x):
    i = plsc.bitcast(x, jnp.int32)
    i = (jnp.full((_L,), _RSQRT_MAGIC, jnp.int32)
         - lax.shift_right_logical(i, jnp.full((_L,), 1, jnp.int32)))
    r = plsc.bitcast(i, jnp.float32)
    half_x = x * 0.5
    for _ in range(3):
        r = r * (1.5 - half_x * r * r)
    return x * r


def _sc_distances(features, class_ids3, centers):
    mesh = plsc.VectorSubcoreMesh(core_axis_name="c", subcore_axis_name="s")

    @functools.partial(
        pl.kernel,
        mesh=mesh,
        compiler_params=pltpu.CompilerParams(needs_layout_passes=False),
        out_type=jax.ShapeDtypeStruct((_B,), jnp.float32),
        scratch_types=[
            pltpu.VMEM((_NCHUNK, _CH), jnp.int32),
            pltpu.VMEM((_CH, _D), jnp.float32),
            pltpu.VMEM((_CH, _D), jnp.float32),
            pltpu.VMEM((_CH * _L,), jnp.float32),
            pltpu.VMEM((_BPW,), jnp.float32),
            pltpu.SemaphoreType.DMA,
        ],
    )
    def k(feat_hbm, idx_hbm, tab_hbm, out_hbm, idx_v, cen_v, feat_v, ps_v,
          out_v, sem):
        cid = lax.axis_index("c")
        sid = lax.axis_index("s")
        wid = sid * _NC + cid
        base = wid * _BPW

        pltpu.sync_copy(idx_hbm.at[wid], idx_v)

        lanes = lax.broadcasted_iota(jnp.int32, (_L,), 0)

        for ci in range(_NCHUNK):
            row0 = base + ci * _CH
            gather = pltpu.async_copy(tab_hbm.at[idx_v.at[ci]], cen_v, sem)
            pltpu.sync_copy(feat_hbm.at[pl.ds(row0, _CH)], feat_v)
            gather.wait()

            def row_body(r, carry):
                acc0 = jnp.zeros((_L,), jnp.float32)
                acc1 = jnp.zeros((_L,), jnp.float32)
                for j in range(_D // (2 * _L)):
                    o = j * 2 * _L
                    d0 = feat_v[r, pl.ds(o, _L)] - cen_v[r, pl.ds(o, _L)]
                    d1 = (feat_v[r, pl.ds(o + _L, _L)]
                          - cen_v[r, pl.ds(o + _L, _L)])
                    acc0 = acc0 + d0 * d0
                    acc1 = acc1 + d1 * d1
                ps_v[pl.ds(r * _L, _L)] = acc0 + acc1
                return carry

            lax.fori_loop(0, _CH, row_body, 0)

            for g in range(_CH // _L):
                addr0 = (g * _L + lanes) * _L
                acc = jnp.zeros((_L,), jnp.float32)
                for j in range(_L):
                    acc = acc + plsc.load_gather(ps_v, [addr0 + j])
                out_v[pl.ds(ci * _CH + g * _L, _L)] = (
                    _vsqrt(acc) * jnp.float32(1.0 / _D))

        pltpu.sync_copy(out_v, out_hbm.at[pl.ds(base, _BPW)])

    return k(features, class_ids3, centers)


def kernel(features, class_ids, centers):
    class_ids3 = class_ids.reshape(_NW, _NCHUNK, _CH)
    dists = _sc_distances(features, class_ids3, centers)
    return dists.reshape(_B, 1)


# speedup vs baseline: 1.2384x; 1.2384x over previous
import functools

import jax
import jax.numpy as jnp
from jax import lax
from jax.experimental import pallas as pl
from jax.experimental.pallas import tpu as pltpu
from jax.experimental.pallas import tpu_sc as plsc

_B = 16384
_D = 128
_NC = 2
_NS = 16
_L = 16
_NW = _NC * _NS
_BPW = _B // _NW
_CH = 128
_NCHUNK = _BPW // _CH

_RSQRT_MAGIC = 0x5F3759DF


def _vsqrt(x):
    i = plsc.bitcast(x, jnp.int32)
    i = (jnp.full((_L,), _RSQRT_MAGIC, jnp.int32)
         - lax.shift_right_logical(i, jnp.full((_L,), 1, jnp.int32)))
    r = plsc.bitcast(i, jnp.float32)
    half_x = x * 0.5
    for _ in range(3):
        r = r * (1.5 - half_x * r * r)
    return x * r


def _sc_distances(features, class_ids3, centers):
    mesh = plsc.VectorSubcoreMesh(core_axis_name="c", subcore_axis_name="s")

    @functools.partial(
        pl.kernel,
        mesh=mesh,
        compiler_params=pltpu.CompilerParams(needs_layout_passes=False),
        out_type=jax.ShapeDtypeStruct((_B,), jnp.float32),
        scratch_types=[
            pltpu.VMEM((_NCHUNK, _CH), jnp.int32),
            pltpu.VMEM((2, _CH, _D), jnp.float32),
            pltpu.VMEM((2, _CH, _D), jnp.float32),
            pltpu.VMEM((_CH * _L,), jnp.float32),
            pltpu.VMEM((_BPW,), jnp.float32),
            pltpu.SemaphoreType.DMA,
            pltpu.SemaphoreType.DMA,
            pltpu.SemaphoreType.DMA,
            pltpu.SemaphoreType.DMA,
        ],
    )
    def k(feat_hbm, idx_hbm, tab_hbm, out_hbm, idx_v, cen_v, feat_v, ps_v,
          out_v, sg0, sg1, sf0, sf1):
        cid = lax.axis_index("c")
        sid = lax.axis_index("s")
        wid = sid * _NC + cid
        base = wid * _BPW
        sgs = (sg0, sg1)
        sfs = (sf0, sf1)

        pltpu.sync_copy(idx_hbm.at[wid], idx_v)

        def launch(ci):
            b = ci & 1
            hg = pltpu.async_copy(tab_hbm.at[idx_v.at[ci]], cen_v.at[b],
                                  sgs[b])
            hf = pltpu.async_copy(feat_hbm.at[pl.ds(base + ci * _CH, _CH)],
                                  feat_v.at[b], sfs[b])
            return hg, hf

        lanes = lax.broadcasted_iota(jnp.int32, (_L,), 0)
        pending = launch(0)

        for ci in range(_NCHUNK):
            if ci + 1 < _NCHUNK:
                nxt = launch(ci + 1)
            hg, hf = pending
            hg.wait()
            hf.wait()
            b = ci & 1

            @plsc.parallel_loop(0, _CH, step=1, unroll=4)
            def _row(r):
                acc0 = jnp.zeros((_L,), jnp.float32)
                acc1 = jnp.zeros((_L,), jnp.float32)
                for j in range(_D // (2 * _L)):
                    o = j * 2 * _L
                    d0 = feat_v[b, r, pl.ds(o, _L)] - cen_v[b, r, pl.ds(o, _L)]
                    d1 = (feat_v[b, r, pl.ds(o + _L, _L)]
                          - cen_v[b, r, pl.ds(o + _L, _L)])
                    acc0 = acc0 + d0 * d0
                    acc1 = acc1 + d1 * d1
                ps_v[pl.ds(r * _L, _L)] = acc0 + acc1

            for g in range(_CH // _L):
                addr0 = (g * _L + lanes) * _L
                acc = [jnp.zeros((_L,), jnp.float32) for _ in range(4)]
                for j in range(_L):
                    acc[j & 3] = acc[j & 3] + plsc.load_gather(
                        ps_v, [addr0 + j])
                total = (acc[0] + acc[1]) + (acc[2] + acc[3])
                out_v[pl.ds(ci * _CH + g * _L, _L)] = (
                    _vsqrt(total) * jnp.float32(1.0 / _D))

            if ci + 1 < _NCHUNK:
                pending = nxt

        pltpu.sync_copy(out_v, out_hbm.at[pl.ds(base, _BPW)])

    return k(features, class_ids3, centers)


def kernel(features, class_ids, centers):
    class_ids3 = class_ids.reshape(_NW, _NCHUNK, _CH)
    dists = _sc_distances(features, class_ids3, centers)
    return dists.reshape(_B, 1)
